# Initial kernel scaffold; baseline (speedup 1.0000x reference)
import jax
import jax.numpy as jnp
from jax.experimental import pallas as pl


def kernel(h_target, h_neighbors, W, a, Wg, bg):
    raise NotImplementedError("write your pallas kernel here")


# speedup vs baseline: 2.3677x; 2.3677x over previous
import jax
import jax.numpy as jnp
from jax.experimental import pallas as pl
from jax.experimental.pallas import tpu as pltpu

N, K, D = 10000, 32, 128
BN = 400


def _asagnn_block(ht_ref, hn_ref, w_ref, a1_ref, a2_ref, wg1_ref, wg2_ref,
                  bg_ref, out_ref):
    ht = ht_ref[...]
    hn = hn_ref[...]
    w = w_ref[...]

    wht = jnp.dot(ht, w, preferred_element_type=jnp.float32)
    whn = jnp.dot(hn.reshape(K * BN, D), w,
                  preferred_element_type=jnp.float32).reshape(K, BN, D)

    e_t = jnp.sum(wht * a1_ref[...], axis=-1, keepdims=True)
    e_n = jnp.sum(whn * a2_ref[...][None], axis=-1, keepdims=True)
    e = e_t[None] + e_n
    e = jnp.where(e >= 0, e, 0.2 * e)
    m = jnp.max(e, axis=0, keepdims=True)
    p = jnp.exp(e - m)
    alpha = p / jnp.sum(p, axis=0, keepdims=True)

    h_agg = jnp.sum(alpha * whn, axis=0)

    glin = (jnp.dot(ht, wg1_ref[...], preferred_element_type=jnp.float32)
            + jnp.dot(h_agg, wg2_ref[...], preferred_element_type=jnp.float32)
            + bg_ref[...])
    gate = jax.nn.sigmoid(glin)
    out_ref[...] = gate * ht + (1.0 - gate) * h_agg


def kernel(h_target, h_neighbors, W, a, Wg, bg):
    a1 = a[:D].reshape(1, D)
    a2 = a[D:].reshape(1, D)
    wg1 = Wg[:D]
    wg2 = Wg[D:]
    bg2 = bg.reshape(1, D)

    grid = (N // BN,)
    return pl.pallas_call(
        _asagnn_block,
        grid=grid,
        in_specs=[
            pl.BlockSpec((BN, D), lambda i: (i, 0)),
            pl.BlockSpec((K, BN, D), lambda i: (0, i, 0)),
            pl.BlockSpec((D, D), lambda i: (0, 0)),
            pl.BlockSpec((1, D), lambda i: (0, 0)),
            pl.BlockSpec((1, D), lambda i: (0, 0)),
            pl.BlockSpec((D, D), lambda i: (0, 0)),
            pl.BlockSpec((D, D), lambda i: (0, 0)),
            pl.BlockSpec((1, D), lambda i: (0, 0)),
        ],
        out_specs=pl.BlockSpec((BN, D), lambda i: (i, 0)),
        out_shape=jax.ShapeDtypeStruct((N, D), jnp.float32),
        compiler_params=pltpu.CompilerParams(
            dimension_semantics=("parallel",),
        ),
    )(h_target, h_neighbors, W, a1, a2, wg1, wg2, bg2)
